# baseline (speedup 1.0000x reference)
import jax
import jax.numpy as jnp
from jax.experimental import pallas as pl
from jax.experimental.pallas import tpu as pltpu

B, L, N = 16, 96, 207
D, DFF = 16, 64
LAYERS, E, K = 3, 4, 2
PATCH = [8, 6, 4, 2]
PRED = 96
NP = 208


def _fwd_kernel(x_ref, sw_ref, sb_ref, slw_ref, slb_ref, wg_ref,
                ew1_ref, eb1_ref, ew2_ref, eb2_ref, pw_ref, pb_ref,
                o_ref, out_s, comb_s):
    xb = x_ref[0]
    nmask = (jax.lax.broadcasted_iota(jnp.int32, (1, NP), 1) < N)
    nmaskf = nmask.astype(jnp.float32)

    mean = jnp.sum(xb, axis=0, keepdims=True) * (1.0 / L)
    xc = xb - mean * nmaskf
    var = jnp.sum(xc * xc, axis=0, keepdims=True) * (1.0 / L)
    stdev = jnp.sqrt(var + 1e-5)
    xn = xc / stdev

    sw = sw_ref[0].reshape(1, 1, D)
    sb = sb_ref[0].reshape(1, 1, D)
    out_s[...] = xn[:, :, None] * sw + sb

    for l in range(LAYERS):
        cur = out_s[...]
        slw = slw_ref[l].reshape(1, 1, D)
        t = jnp.sum(cur * slw, axis=2)
        gm = jnp.sum(t * nmaskf, axis=1, keepdims=True) * (1.0 / N) \
            + slb_ref[0, l]
        logits = jnp.sum(gm * wg_ref[l], axis=0, keepdims=True)

        eio = jax.lax.broadcasted_iota(jnp.int32, (1, E), 1)
        ls = [jnp.sum(jnp.where(eio == e, logits, 0.0)) for e in range(E)]
        m1 = jnp.maximum(jnp.maximum(ls[0], ls[1]),
                         jnp.maximum(ls[2], ls[3]))
        sel1 = []
        taken = None
        for e in range(E):
            hit = ls[e] == m1
            if taken is None:
                sel1.append(hit)
                taken = hit
            else:
                sel1.append(jnp.logical_and(hit, jnp.logical_not(taken)))
                taken = jnp.logical_or(taken, hit)
        rs = [jnp.where(sel1[e], jnp.float32(-1e30), ls[e]) for e in range(E)]
        m2 = jnp.maximum(jnp.maximum(rs[0], rs[1]),
                         jnp.maximum(rs[2], rs[3]))
        sel2 = []
        taken = None
        for e in range(E):
            hit = rs[e] == m2
            if taken is None:
                sel2.append(hit)
                taken = hit
            else:
                sel2.append(jnp.logical_and(hit, jnp.logical_not(taken)))
                taken = jnp.logical_or(taken, hit)
        d21 = jnp.exp(m2 - m1)
        g1 = 1.0 / (1.0 + d21)
        g2 = d21 / (1.0 + d21)
        gates = [jnp.where(sel1[e], g1, 0.0) + jnp.where(sel2[e], g2, 0.0)
                 for e in range(E)]

        comb_s[...] = jnp.zeros((L, NP, D), jnp.float32)
        for e in range(E):
            p = PATCH[e]
            gate = gates[e]
            w1 = ew1_ref[l, e]
            b1 = eb1_ref[l, e].reshape(1, DFF)
            w2 = ew2_ref[l, e]
            b2 = eb2_ref[l, e].reshape(1, D)

            @pl.when(gate > 0.0)
            def _(cur=cur, p=p, gate=gate, w1=w1, b1=b1, w2=w2, b2=b2):
                x4 = cur.reshape(L // p, p, NP, D)
                ctx = jnp.mean(x4, axis=1, keepdims=True)
                xe = (x4 + ctx).reshape(L * NP, D)
                h = jnp.maximum(jnp.dot(xe, w1,
                                        preferred_element_type=jnp.float32)
                                + b1, 0.0)
                y = jnp.dot(h, w2,
                            preferred_element_type=jnp.float32) + b2
                comb_s[...] += gate * y.reshape(L, NP, D)

        out_s[...] = cur + comb_s[...]

    last = out_s[:, N - 1, :]
    prod = last[:, :, None] * pw_ref[...]
    y = jnp.sum(jnp.sum(prod, axis=0), axis=0, keepdims=True) \
        + pb_ref[...]
    res = y * stdev[0, N - 1] + mean[0, N - 1]
    o_ref[...] = jnp.broadcast_to(res[None], (1, 8, PRED))


def kernel(x, start_w, start_b, sl_w, sl_b, wg, wn, ew1, eb1, ew2, eb2,
           proj_w, proj_b):
    del wn
    xp = jnp.pad(x, ((0, 0), (0, 0), (0, NP - N)))
    full = lambda shape: pl.BlockSpec(shape, lambda b: (0,) * len(shape))
    return pl.pallas_call(
        _fwd_kernel,
        grid=(B,),
        in_specs=[
            pl.BlockSpec((1, L, NP), lambda b: (b, 0, 0)),
            full((1, D)),
            full((1, D)),
            full((LAYERS, D)),
            full((1, LAYERS)),
            full((LAYERS, L, E)),
            full((LAYERS, E, D, DFF)),
            full((LAYERS, E, DFF)),
            full((LAYERS, E, DFF, D)),
            full((LAYERS, E, D)),
            full((L, D, PRED)),
            full((1, PRED)),
        ],
        out_specs=pl.BlockSpec((1, 8, PRED), lambda b: (b, 0, 0)),
        out_shape=jax.ShapeDtypeStruct((B, 8, PRED), jnp.float32),
        scratch_shapes=[
            pltpu.VMEM((L, NP, D), jnp.float32),
            pltpu.VMEM((L, NP, D), jnp.float32),
        ],
    )(xp, start_w, start_b.reshape(1, D), sl_w, sl_b.reshape(1, LAYERS),
      wg, ew1, eb1, ew2, eb2, proj_w.reshape(L, D, PRED),
      proj_b.reshape(1, PRED))[:, 0, :]


# speedup vs baseline: 1.1487x; 1.1487x over previous
import jax
import jax.numpy as jnp
from jax.experimental import pallas as pl
from jax.experimental.pallas import tpu as pltpu

B, L, N = 16, 96, 207
D, DFF = 16, 64
LAYERS, E, K = 3, 4, 2
PATCH = [8, 6, 4, 2]
PRED = 96
NP = 208


def _fwd_kernel(x_ref, sw_ref, sb_ref, slw_ref, slb_ref, wg_ref,
                ew1_ref, eb1_ref, ew2_ref, eb2_ref, pw_ref, pb_ref,
                o_ref, out_s):
    xb = x_ref[0]
    nmask = (jax.lax.broadcasted_iota(jnp.int32, (1, NP), 1) < N)
    nmaskf = nmask.astype(jnp.float32)

    mean = jnp.sum(xb, axis=0, keepdims=True) * (1.0 / L)
    xc = xb - mean * nmaskf
    var = jnp.sum(xc * xc, axis=0, keepdims=True) * (1.0 / L)
    stdev = jnp.sqrt(var + 1e-5)
    xn = xc / stdev

    sw = sw_ref[0].reshape(1, 1, D)
    sb = sb_ref[0].reshape(1, 1, D)
    out_s[...] = xn[:, :, None] * sw + sb

    for l in range(LAYERS):
        cur = out_s[...]
        slw = slw_ref[l].reshape(1, 1, D)
        t = jnp.sum(cur * slw, axis=2)
        gm = jnp.sum(t * nmaskf, axis=1, keepdims=True) * (1.0 / N) \
            + slb_ref[0, l]
        logits = jnp.sum(gm * wg_ref[l], axis=0, keepdims=True)

        eio = jax.lax.broadcasted_iota(jnp.int32, (1, E), 1)
        ls = [jnp.sum(jnp.where(eio == e, logits, 0.0)) for e in range(E)]
        m1 = jnp.maximum(jnp.maximum(ls[0], ls[1]),
                         jnp.maximum(ls[2], ls[3]))
        sel1 = []
        taken = None
        for e in range(E):
            hit = ls[e] == m1
            if taken is None:
                sel1.append(hit)
                taken = hit
            else:
                sel1.append(jnp.logical_and(hit, jnp.logical_not(taken)))
                taken = jnp.logical_or(taken, hit)
        rs = [jnp.where(sel1[e], jnp.float32(-1e30), ls[e]) for e in range(E)]
        m2 = jnp.maximum(jnp.maximum(rs[0], rs[1]),
                         jnp.maximum(rs[2], rs[3]))
        sel2 = []
        taken = None
        for e in range(E):
            hit = rs[e] == m2
            if taken is None:
                sel2.append(hit)
                taken = hit
            else:
                sel2.append(jnp.logical_and(hit, jnp.logical_not(taken)))
                taken = jnp.logical_or(taken, hit)
        d21 = jnp.exp(m2 - m1)
        g1 = 1.0 / (1.0 + d21)
        g2 = d21 / (1.0 + d21)
        gates = [jnp.where(sel1[e], g1, 0.0) + jnp.where(sel2[e], g2, 0.0)
                 for e in range(E)]

        for e in range(E):
            p = PATCH[e]
            gate = gates[e]
            w1 = ew1_ref[l, e]
            b1 = eb1_ref[l, e].reshape(1, DFF)
            w2 = ew2_ref[l, e]
            b2 = eb2_ref[l, e].reshape(1, D)

            @pl.when(gate > 0.0)
            def _(cur=cur, p=p, gate=gate, w1=w1, b1=b1, w2=w2, b2=b2):
                x4 = cur.reshape(L // p, p, NP, D)
                ctx = jnp.mean(x4, axis=1, keepdims=True)
                xe = (x4 + ctx).reshape(L * NP, D)
                h = jnp.maximum(jnp.dot(xe, w1,
                                        preferred_element_type=jnp.float32)
                                + b1, 0.0)
                y = jnp.dot(h, w2,
                            preferred_element_type=jnp.float32) + b2
                out_s[...] += gate * y.reshape(L, NP, D)

    last = out_s[:, N - 1, :]
    prod = last[:, :, None] * pw_ref[...]
    y = jnp.sum(jnp.sum(prod, axis=0), axis=0, keepdims=True) \
        + pb_ref[...]
    res = y * stdev[0, N - 1] + mean[0, N - 1]
    o_ref[...] = jnp.broadcast_to(res[None], (1, 8, PRED))


def kernel(x, start_w, start_b, sl_w, sl_b, wg, wn, ew1, eb1, ew2, eb2,
           proj_w, proj_b):
    del wn
    xp = jnp.pad(x, ((0, 0), (0, 0), (0, NP - N)))
    full = lambda shape: pl.BlockSpec(shape, lambda b: (0,) * len(shape))
    return pl.pallas_call(
        _fwd_kernel,
        grid=(B,),
        in_specs=[
            pl.BlockSpec((1, L, NP), lambda b: (b, 0, 0)),
            full((1, D)),
            full((1, D)),
            full((LAYERS, D)),
            full((1, LAYERS)),
            full((LAYERS, L, E)),
            full((LAYERS, E, D, DFF)),
            full((LAYERS, E, DFF)),
            full((LAYERS, E, DFF, D)),
            full((LAYERS, E, D)),
            full((L, D, PRED)),
            full((1, PRED)),
        ],
        out_specs=pl.BlockSpec((1, 8, PRED), lambda b: (b, 0, 0)),
        out_shape=jax.ShapeDtypeStruct((B, 8, PRED), jnp.float32),
        scratch_shapes=[
            pltpu.VMEM((L, NP, D), jnp.float32),
        ],
    )(xp, start_w, start_b.reshape(1, D), sl_w, sl_b.reshape(1, LAYERS),
      wg, ew1, eb1, ew2, eb2, proj_w.reshape(L, D, PRED),
      proj_b.reshape(1, PRED))[:, 0, :]


# speedup vs baseline: 1.7896x; 1.5579x over previous
import jax
import jax.numpy as jnp
from jax.experimental import pallas as pl
from jax.experimental.pallas import tpu as pltpu

B, L, N = 16, 96, 207
D, DFF = 16, 64
LAYERS, E, K = 3, 4, 2
PATCH = [8, 6, 4, 2]
PRED = 96
NP = 208


def _fwd_kernel(x_ref, sw_ref, sb_ref, gw_ref, slb_ref, wg_ref,
                ew1_ref, eb1_ref, ew2_ref, eb2_ref, pw_ref, pb_ref,
                o_ref, out_s):
    xb = x_ref[0]
    nmask = (jax.lax.broadcasted_iota(jnp.int32, (1, NP), 1) < N)
    nmaskf = nmask.astype(jnp.float32)

    mean = jnp.sum(xb, axis=0, keepdims=True) * (1.0 / L)
    xc = xb - mean * nmaskf
    var = jnp.sum(xc * xc, axis=0, keepdims=True) * (1.0 / L)
    stdev = jnp.sqrt(var + 1e-5)
    xn = xc / stdev

    sw = sw_ref[0].reshape(1, 1, D)
    sb = sb_ref[0].reshape(1, 1, D)
    out_s[...] = xn[:, :, None] * sw + sb

    for l in range(LAYERS):
        cur = out_s[...]
        gm = jnp.sum(cur * gw_ref[l][None], axis=(1, 2),
                     keepdims=True) * (1.0 / N) + slb_ref[0, l]
        logits = jnp.sum(gm * wg_ref[l][:, None, :], axis=0)

        eio = jax.lax.broadcasted_iota(jnp.int32, (1, E), 1)
        ls = [jnp.sum(jnp.where(eio == e, logits, 0.0)) for e in range(E)]
        m1 = jnp.maximum(jnp.maximum(ls[0], ls[1]),
                         jnp.maximum(ls[2], ls[3]))
        sel1 = []
        taken = None
        for e in range(E):
            hit = ls[e] == m1
            if taken is None:
                sel1.append(hit)
                taken = hit
            else:
                sel1.append(jnp.logical_and(hit, jnp.logical_not(taken)))
                taken = jnp.logical_or(taken, hit)
        rs = [jnp.where(sel1[e], jnp.float32(-1e30), ls[e]) for e in range(E)]
        m2 = jnp.maximum(jnp.maximum(rs[0], rs[1]),
                         jnp.maximum(rs[2], rs[3]))
        sel2 = []
        taken = None
        for e in range(E):
            hit = rs[e] == m2
            if taken is None:
                sel2.append(hit)
                taken = hit
            else:
                sel2.append(jnp.logical_and(hit, jnp.logical_not(taken)))
                taken = jnp.logical_or(taken, hit)
        d21 = jnp.exp(m2 - m1)
        g1 = 1.0 / (1.0 + d21)
        g2 = d21 / (1.0 + d21)
        gates = [jnp.where(sel1[e], g1, 0.0) + jnp.where(sel2[e], g2, 0.0)
                 for e in range(E)]

        for e in range(E):
            p = PATCH[e]
            gate = gates[e]
            w1 = ew1_ref[l, e]
            b1 = eb1_ref[l, e].reshape(1, DFF)
            w2 = ew2_ref[l, e]
            b2 = eb2_ref[l, e].reshape(1, D)

            @pl.when(gate > 0.0)
            def _(cur=cur, p=p, gate=gate, w1=w1, b1=b1, w2=w2, b2=b2):
                x4 = cur.reshape(L // p, p, NP, D)
                ctx = jnp.mean(x4, axis=1, keepdims=True)
                xe = (x4 + ctx).reshape(L * NP, D)
                h = jnp.maximum(jnp.dot(xe, w1,
                                        preferred_element_type=jnp.float32)
                                + b1, 0.0)
                y = jnp.dot(h, w2,
                            preferred_element_type=jnp.float32) + b2
                out_s[...] += gate * y.reshape(L, NP, D)

    last = out_s[:, N - 1, :]
    prod = last[:, :, None] * pw_ref[...]
    y = jnp.sum(jnp.sum(prod, axis=0), axis=0, keepdims=True) \
        + pb_ref[...]
    res = y * stdev[0, N - 1] + mean[0, N - 1]
    o_ref[...] = jnp.broadcast_to(res[None], (1, 8, PRED))


def kernel(x, start_w, start_b, sl_w, sl_b, wg, wn, ew1, eb1, ew2, eb2,
           proj_w, proj_b):
    del wn
    xp = jnp.pad(x, ((0, 0), (0, 0), (0, NP - N)))
    full = lambda shape: pl.BlockSpec(shape, lambda b: (0,) * len(shape))
    pcall = pl.pallas_call(
        _fwd_kernel,
        grid=(B,),
        in_specs=[
            pl.BlockSpec((1, L, NP), lambda b: (b, 0, 0)),
            full((1, D)),
            full((1, D)),
            full((LAYERS, NP, D)),
            full((1, LAYERS)),
            full((LAYERS, L, E)),
            full((LAYERS, E, D, DFF)),
            full((LAYERS, E, DFF)),
            full((LAYERS, E, DFF, D)),
            full((LAYERS, E, D)),
            full((L, D, PRED)),
            full((1, PRED)),
        ],
        out_specs=pl.BlockSpec((1, 8, PRED), lambda b: (b, 0, 0)),
        out_shape=jax.ShapeDtypeStruct((B, 8, PRED), jnp.float32),
        scratch_shapes=[
            pltpu.VMEM((L, NP, D), jnp.float32),
        ],
    )
    nmask_h = (jnp.arange(NP) < N).astype(jnp.float32)
    gw = sl_w[:, None, :] * nmask_h[None, :, None]
    return pcall(xp, start_w, start_b.reshape(1, D), gw,
      sl_b.reshape(1, LAYERS),
      wg, ew1, eb1, ew2, eb2, proj_w.reshape(L, D, PRED),
      proj_b.reshape(1, PRED))[:, 0, :]


# speedup vs baseline: 1.8071x; 1.0098x over previous
import jax
import jax.numpy as jnp
from jax.experimental import pallas as pl
from jax.experimental.pallas import tpu as pltpu

B, L, N = 16, 96, 207
D, DFF = 16, 64
LAYERS, E, K = 3, 4, 2
PATCH = [8, 6, 4, 2]
PRED = 96
NP = 208


def _fwd_kernel(x_ref, sw_ref, sb_ref, gw_ref, slb_ref, wg_ref,
                ew1_ref, eb1_ref, ew2_ref, eb2_ref, pw_ref, pb_ref,
                o_ref, out_s):
    xb = x_ref[0]
    nmask = (jax.lax.broadcasted_iota(jnp.int32, (1, NP), 1) < N)
    nmaskf = nmask.astype(jnp.float32)

    mean = jnp.sum(xb, axis=0, keepdims=True) * (1.0 / L)
    xc = xb - mean * nmaskf
    var = jnp.sum(xc * xc, axis=0, keepdims=True) * (1.0 / L)
    stdev = jnp.sqrt(var + 1e-5)
    xn = xc / stdev

    sw = sw_ref[0].reshape(1, 1, D)
    sb = sb_ref[0].reshape(1, 1, D)
    out_s[...] = xn[:, :, None] * sw + sb

    for l in range(LAYERS):
        cur = out_s[...]
        cur2d = cur.reshape(L * NP, D)
        gm = jnp.sum(cur * gw_ref[l][None], axis=(1, 2),
                     keepdims=True) * (1.0 / N) + slb_ref[0, l]
        logits = jnp.sum(gm * wg_ref[l][:, None, :], axis=0)

        eio = jax.lax.broadcasted_iota(jnp.int32, (1, E), 1)
        ls = [jnp.sum(jnp.where(eio == e, logits, 0.0)) for e in range(E)]
        m1 = jnp.maximum(jnp.maximum(ls[0], ls[1]),
                         jnp.maximum(ls[2], ls[3]))
        sel1 = []
        taken = None
        for e in range(E):
            hit = ls[e] == m1
            if taken is None:
                sel1.append(hit)
                taken = hit
            else:
                sel1.append(jnp.logical_and(hit, jnp.logical_not(taken)))
                taken = jnp.logical_or(taken, hit)
        rs = [jnp.where(sel1[e], jnp.float32(-1e30), ls[e]) for e in range(E)]
        m2 = jnp.maximum(jnp.maximum(rs[0], rs[1]),
                         jnp.maximum(rs[2], rs[3]))
        sel2 = []
        taken = None
        for e in range(E):
            hit = rs[e] == m2
            if taken is None:
                sel2.append(hit)
                taken = hit
            else:
                sel2.append(jnp.logical_and(hit, jnp.logical_not(taken)))
                taken = jnp.logical_or(taken, hit)
        d21 = jnp.exp(m2 - m1)
        g1 = 1.0 / (1.0 + d21)
        g2 = d21 / (1.0 + d21)
        gates = [jnp.where(sel1[e], g1, 0.0) + jnp.where(sel2[e], g2, 0.0)
                 for e in range(E)]

        for e in range(E):
            p = PATCH[e]
            gate = gates[e]
            w1 = ew1_ref[l, e]
            b1 = eb1_ref[l, e].reshape(1, DFF)
            w2 = ew2_ref[l, e]
            b2 = eb2_ref[l, e].reshape(1, D)

            @pl.when(gate > 0.0)
            def _(cur2d=cur2d, p=p, gate=gate, w1=w1, b1=b1, w2=w2, b2=b2):
                z = jnp.dot(cur2d, w1, preferred_element_type=jnp.float32)
                z4 = z.reshape(L // p, p, NP, DFF)
                h4 = jnp.maximum(z4 + jnp.mean(z4, axis=1, keepdims=True)
                                 + b1[None, None], 0.0)
                y = jnp.dot(h4.reshape(L * NP, DFF), gate * w2,
                            preferred_element_type=jnp.float32) + gate * b2
                out_s[...] += y.reshape(L, NP, D)

    last = out_s[:, N - 1, :]
    prod = last[:, :, None] * pw_ref[...]
    y = jnp.sum(jnp.sum(prod, axis=0), axis=0, keepdims=True) \
        + pb_ref[...]
    res = y * stdev[0, N - 1] + mean[0, N - 1]
    o_ref[...] = jnp.broadcast_to(res[None], (1, 8, PRED))


def kernel(x, start_w, start_b, sl_w, sl_b, wg, wn, ew1, eb1, ew2, eb2,
           proj_w, proj_b):
    del wn
    xp = jnp.pad(x, ((0, 0), (0, 0), (0, NP - N)))
    full = lambda shape: pl.BlockSpec(shape, lambda b: (0,) * len(shape))
    pcall = pl.pallas_call(
        _fwd_kernel,
        grid=(B,),
        in_specs=[
            pl.BlockSpec((1, L, NP), lambda b: (b, 0, 0)),
            full((1, D)),
            full((1, D)),
            full((LAYERS, NP, D)),
            full((1, LAYERS)),
            full((LAYERS, L, E)),
            full((LAYERS, E, D, DFF)),
            full((LAYERS, E, DFF)),
            full((LAYERS, E, DFF, D)),
            full((LAYERS, E, D)),
            full((L, D, PRED)),
            full((1, PRED)),
        ],
        out_specs=pl.BlockSpec((1, 8, PRED), lambda b: (b, 0, 0)),
        out_shape=jax.ShapeDtypeStruct((B, 8, PRED), jnp.float32),
        scratch_shapes=[
            pltpu.VMEM((L, NP, D), jnp.float32),
        ],
    )
    nmask_h = (jnp.arange(NP) < N).astype(jnp.float32)
    gw = sl_w[:, None, :] * nmask_h[None, :, None]
    return pcall(xp, start_w, start_b.reshape(1, D), gw,
      sl_b.reshape(1, LAYERS),
      wg, ew1, eb1, ew2, eb2, proj_w.reshape(L, D, PRED),
      proj_b.reshape(1, PRED))[:, 0, :]
